# baseline (speedup 1.0000x reference)
import functools

import jax
import jax.numpy as jnp
from jax import lax
from jax.experimental import pallas as pl
from jax.experimental.pallas import tpu as pltpu


def _silu(v):
    return v * jax.nn.sigmoid(v)


def _dott(w, v):
    return lax.dot_general(w, v, (((0,), (0,)), ((), ())),
                           preferred_element_type=jnp.float32)


def _csp1_body(C_, H, W, n_res,
               x_ref, wub_ref, wra_ref, wrb_ref, wtie_ref, sb_ref,
               out_ref):
    M = x_ref.shape[1]

    sb = sb_ref[...]
    b_ub = sb[0 * C_:2 * C_]
    b_ra = sb[2 * C_:3 * C_]
    b_rb = sb[3 * C_:4 * C_]
    stie_u = sb[4 * C_:5 * C_]
    btie_u = sb[5 * C_:6 * C_]

    x = x_ref[...].astype(jnp.bfloat16)

    yb = _dott(wub_ref[...], x) + b_ub
    y = _silu(yb[:C_]).astype(jnp.bfloat16)
    bot_tt = yb[C_:]

    pix = lax.broadcasted_iota(jnp.int32, (1, M), 1)
    i_idx = (pix // W) % H
    j_idx = pix % W
    row_ok = {-1: i_idx >= 1, 1: i_idx < (H - 1)}
    col_ok = {-1: j_idx >= 1, 1: j_idx < (W - 1)}
    tap_mask = {}
    for r in (-1, 0, 1):
        for c in (-1, 0, 1):
            if r == 0 and c == 0:
                msk = None
            elif r == 0:
                msk = col_ok[c]
            elif c == 0:
                msk = row_ok[r]
            else:
                msk = row_ok[r] & col_ok[c]
            tap_mask[(r, c)] = msk

    zero_bf = jnp.zeros((), jnp.bfloat16)
    cur = y
    for _ in range(n_res):
        tb = _silu(_dott(wra_ref[...], cur) + b_ra).astype(jnp.bfloat16)
        acc = b_rb
        for gi, r in enumerate((-1, 0, 1)):
            taps = []
            for c in (-1, 0, 1):
                off = r * W + c
                sh = tb if off == 0 else pltpu.roll(tb, shift=(-off) % M, axis=1)
                msk = tap_mask[(r, c)]
                if msk is not None:
                    sh = jnp.where(msk, sh, zero_bf)
                taps.append(sh)
            grp = jnp.concatenate(taps, axis=0)
            acc = acc + _dott(wrb_ref.at[3 * C_ * gi:3 * C_ * (gi + 1)][...], grp)
        cur = _silu(acc).astype(jnp.bfloat16)
    up = (y + cur).astype(jnp.float32)

    tt = jnp.concatenate([up * stie_u + btie_u, bot_tt], axis=0)
    tt = jnp.where(tt >= 0, tt, 0.01 * tt).astype(jnp.bfloat16)
    out_ref[...] = _dott(wtie_ref[...], tt)


def _full_spec(shape):
    nd = len(shape)
    return pl.BlockSpec(shape, lambda n, _nd=nd: (0,) * _nd)


def kernel(x, w_up1, s_up1, b_up1, w_ra, s_ra, b_ra, w_rb_hwio, s_rb, b_rb,
           w_bot, b_bot, s_tie, b_tie, w_tie):
    n_res = 2
    N, C1, H, W = x.shape
    C_ = w_up1.shape[1]
    C2 = w_tie.shape[1]
    M = H * W

    xf = x.reshape(N, C1, M)

    s_tie = s_tie.reshape(-1)
    b_tie = b_tie.reshape(-1)
    s_tie_u, s_tie_b = s_tie[:C_], s_tie[C_:]
    b_tie_u, b_tie_b = b_tie[:C_], b_tie[C_:]

    wub = jnp.concatenate([
        w_up1 * s_up1.reshape(-1)[None, :],
        w_bot * s_tie_b[None, :],
    ], axis=1).astype(jnp.bfloat16)
    wra = (w_ra * s_ra.reshape(-1)[None, :]).astype(jnp.bfloat16)
    wrb = (w_rb_hwio.reshape(9 * C_, C_)
           * s_rb.reshape(-1)[None, :]).astype(jnp.bfloat16)
    wtie = w_tie.astype(jnp.bfloat16)

    b_bot_f = s_tie_b * b_bot.reshape(-1) + b_tie_b
    sb = jnp.concatenate([
        b_up1.reshape(-1), b_bot_f,
        b_ra.reshape(-1), b_rb.reshape(-1),
        s_tie_u, b_tie_u,
    ]).reshape(-1, 1).astype(jnp.float32)

    body = functools.partial(_csp1_body, C_, H, W, n_res)

    out = pl.pallas_call(
        body,
        out_shape=jax.ShapeDtypeStruct((N, C2, M), jnp.float32),
        grid=(N,),
        in_specs=[
            pl.BlockSpec((None, C1, M), lambda n: (n, 0, 0)),
            _full_spec(wub.shape), _full_spec(wra.shape),
            _full_spec(wrb.shape), _full_spec(wtie.shape), _full_spec(sb.shape),
        ],
        out_specs=pl.BlockSpec((None, C2, M), lambda n: (n, 0, 0)),
        compiler_params=pltpu.CompilerParams(
            dimension_semantics=("parallel",)),
    )(xf, wub, wra, wrb, wtie, sb)

    return out.reshape(N, C2, H, W)


# speedup vs baseline: 1.0236x; 1.0236x over previous
import functools

import jax
import jax.numpy as jnp
from jax import lax
from jax.experimental import pallas as pl
from jax.experimental.pallas import tpu as pltpu


def _silu(v):
    return v * jax.nn.sigmoid(v)


def _dott(w, v):
    return lax.dot_general(w, v, (((0,), (0,)), ((), ())),
                           preferred_element_type=jnp.float32)


def _csp1_body(C_, H, W, n_res,
               x_ref, wub_ref, wra_ref, wrb_ref, wtie_ref, sb_ref,
               out_ref):
    M = x_ref.shape[1]

    sb = sb_ref[...]
    b_ub = sb[0 * C_:2 * C_]
    b_ra = sb[2 * C_:3 * C_]
    b_rb = sb[3 * C_:4 * C_]
    stie_u = sb[4 * C_:5 * C_]
    btie_u = sb[5 * C_:6 * C_]

    x = x_ref[...].astype(jnp.bfloat16)

    yb = _dott(wub_ref[...], x) + b_ub
    y = _silu(yb[:C_]).astype(jnp.bfloat16)
    bot_tt = yb[C_:]

    pix = lax.broadcasted_iota(jnp.int32, (1, M), 1)
    i_idx = (pix // W) % H
    j_idx = pix % W
    row_ok = {-1: i_idx >= 1, 1: i_idx < (H - 1)}
    col_ok = {-1: j_idx >= 1, 1: j_idx < (W - 1)}
    tap_mask = {}
    for r in (-1, 0, 1):
        for c in (-1, 0, 1):
            if r == 0 and c == 0:
                msk = None
            elif r == 0:
                msk = col_ok[c]
            elif c == 0:
                msk = row_ok[r]
            else:
                msk = row_ok[r] & col_ok[c]
            tap_mask[(r, c)] = msk

    zero_bf = jnp.zeros((), jnp.bfloat16)
    cur = y
    for _ in range(n_res):
        tb = _silu(_dott(wra_ref[...], cur) + b_ra).astype(jnp.bfloat16)
        taps = []
        for r in (-1, 0, 1):
            for c in (-1, 0, 1):
                off = r * W + c
                sh = tb if off == 0 else pltpu.roll(tb, shift=(-off) % M, axis=1)
                msk = tap_mask[(r, c)]
                if msk is not None:
                    sh = jnp.where(msk, sh, zero_bf)
                taps.append(sh)
        col = jnp.concatenate(taps, axis=0)
        cur = _silu(_dott(wrb_ref[...], col) + b_rb).astype(jnp.bfloat16)
    up = (y + cur).astype(jnp.float32)

    tt = jnp.concatenate([up * stie_u + btie_u, bot_tt], axis=0)
    tt = jnp.where(tt >= 0, tt, 0.01 * tt).astype(jnp.bfloat16)
    out_ref[...] = _dott(wtie_ref[...], tt)


def _full_spec(shape):
    nd = len(shape)
    return pl.BlockSpec(shape, lambda n, _nd=nd: (0,) * _nd)


def kernel(x, w_up1, s_up1, b_up1, w_ra, s_ra, b_ra, w_rb_hwio, s_rb, b_rb,
           w_bot, b_bot, s_tie, b_tie, w_tie):
    n_res = 2
    N, C1, H, W = x.shape
    C_ = w_up1.shape[1]
    C2 = w_tie.shape[1]
    M = H * W

    xf = x.reshape(N, C1, M)

    s_tie = s_tie.reshape(-1)
    b_tie = b_tie.reshape(-1)
    s_tie_u, s_tie_b = s_tie[:C_], s_tie[C_:]
    b_tie_u, b_tie_b = b_tie[:C_], b_tie[C_:]

    wub = jnp.concatenate([
        w_up1 * s_up1.reshape(-1)[None, :],
        w_bot * s_tie_b[None, :],
    ], axis=1).astype(jnp.bfloat16)
    wra = (w_ra * s_ra.reshape(-1)[None, :]).astype(jnp.bfloat16)
    wrb = (w_rb_hwio.reshape(9 * C_, C_)
           * s_rb.reshape(-1)[None, :]).astype(jnp.bfloat16)
    wtie = w_tie.astype(jnp.bfloat16)

    b_bot_f = s_tie_b * b_bot.reshape(-1) + b_tie_b
    sb = jnp.concatenate([
        b_up1.reshape(-1), b_bot_f,
        b_ra.reshape(-1), b_rb.reshape(-1),
        s_tie_u, b_tie_u,
    ]).reshape(-1, 1).astype(jnp.float32)

    body = functools.partial(_csp1_body, C_, H, W, n_res)

    out = pl.pallas_call(
        body,
        out_shape=jax.ShapeDtypeStruct((N, C2, M), jnp.float32),
        grid=(N,),
        in_specs=[
            pl.BlockSpec((None, C1, M), lambda n: (n, 0, 0)),
            _full_spec(wub.shape), _full_spec(wra.shape),
            _full_spec(wrb.shape), _full_spec(wtie.shape), _full_spec(sb.shape),
        ],
        out_specs=pl.BlockSpec((None, C2, M), lambda n: (n, 0, 0)),
        compiler_params=pltpu.CompilerParams(
            dimension_semantics=("parallel",)),
    )(xf, wub, wra, wrb, wtie, sb)

    return out.reshape(N, C2, H, W)


# speedup vs baseline: 1.0327x; 1.0089x over previous
import functools

import jax
import jax.numpy as jnp
from jax import lax
from jax.experimental import pallas as pl
from jax.experimental.pallas import tpu as pltpu


def _silu(v):
    return v * jax.nn.sigmoid(v)


def _dott(w, v):
    return lax.dot_general(w, v, (((0,), (0,)), ((), ())),
                           preferred_element_type=jnp.float32)


def _csp1_body(C_, H, W, n_res,
               x_ref, wub_ref, wra_ref, wrb_ref, wtie_ref, sb_ref, sc_ref,
               out_ref):
    M = x_ref.shape[1]

    sb = sb_ref[...]
    b_ub = sb[0 * C_:2 * C_]
    b_ra = sb[2 * C_:3 * C_]
    b_rb = sb[3 * C_:4 * C_]
    stie_u = sb[4 * C_:5 * C_]
    btie_u = sb[5 * C_:6 * C_]

    sc = sc_ref[...]
    wub = (wub_ref[...] * sc[0:1]).astype(jnp.bfloat16)
    wra = (wra_ref[...] * sc[1:2, :C_]).astype(jnp.bfloat16)
    wrb = (wrb_ref[...] * sc[2:3, :C_]).astype(jnp.bfloat16)
    wtie = wtie_ref[...].astype(jnp.bfloat16)

    x = x_ref[...].astype(jnp.bfloat16)

    yb = _dott(wub, x) + b_ub
    y = _silu(yb[:C_]).astype(jnp.bfloat16)
    bot_tt = yb[C_:]

    pix = lax.broadcasted_iota(jnp.int32, (1, M), 1)
    i_idx = (pix // W) % H
    j_idx = pix % W
    row_ok = {-1: i_idx >= 1, 1: i_idx < (H - 1)}
    col_ok = {-1: j_idx >= 1, 1: j_idx < (W - 1)}
    tap_mask = {}
    for r in (-1, 0, 1):
        for c in (-1, 0, 1):
            if r == 0 and c == 0:
                msk = None
            elif r == 0:
                msk = col_ok[c]
            elif c == 0:
                msk = row_ok[r]
            else:
                msk = row_ok[r] & col_ok[c]
            tap_mask[(r, c)] = msk

    zero_bf = jnp.zeros((), jnp.bfloat16)
    cur = y
    for _ in range(n_res):
        tb = _silu(_dott(wra, cur) + b_ra).astype(jnp.bfloat16)
        taps = []
        for r in (-1, 0, 1):
            for c in (-1, 0, 1):
                off = r * W + c
                sh = tb if off == 0 else pltpu.roll(tb, shift=(-off) % M, axis=1)
                msk = tap_mask[(r, c)]
                if msk is not None:
                    sh = jnp.where(msk, sh, zero_bf)
                taps.append(sh)
        col = jnp.concatenate(taps, axis=0)
        cur = _silu(_dott(wrb, col) + b_rb).astype(jnp.bfloat16)
    up = (y + cur).astype(jnp.float32)

    tt = jnp.concatenate([up * stie_u + btie_u, bot_tt], axis=0)
    tt = jnp.where(tt >= 0, tt, 0.01 * tt).astype(jnp.bfloat16)
    out_ref[...] = _dott(wtie, tt)


def _full_spec(shape):
    nd = len(shape)
    return pl.BlockSpec(shape, lambda n, _nd=nd: (0,) * _nd)


def kernel(x, w_up1, s_up1, b_up1, w_ra, s_ra, b_ra, w_rb_hwio, s_rb, b_rb,
           w_bot, b_bot, s_tie, b_tie, w_tie):
    n_res = 2
    N, C1, H, W = x.shape
    C_ = w_up1.shape[1]
    C2 = w_tie.shape[1]
    M = H * W

    xf = x.reshape(N, C1, M)

    s_tie = s_tie.reshape(-1)
    b_tie = b_tie.reshape(-1)
    s_tie_u, s_tie_b = s_tie[:C_], s_tie[C_:]
    b_tie_u, b_tie_b = b_tie[:C_], b_tie[C_:]

    wub = jnp.concatenate([w_up1, w_bot], axis=1)
    wrb = w_rb_hwio.reshape(9 * C_, C_)

    b_bot_f = s_tie_b * b_bot.reshape(-1) + b_tie_b
    sb = jnp.concatenate([
        b_up1.reshape(-1), b_bot_f,
        b_ra.reshape(-1), b_rb.reshape(-1),
        s_tie_u, b_tie_u,
    ]).reshape(-1, 1).astype(jnp.float32)
    sc = jnp.stack([
        jnp.concatenate([s_up1.reshape(-1), s_tie_b]),
        jnp.concatenate([s_ra.reshape(-1), s_ra.reshape(-1)]),
        jnp.concatenate([s_rb.reshape(-1), s_rb.reshape(-1)]),
    ]).astype(jnp.float32)

    body = functools.partial(_csp1_body, C_, H, W, n_res)

    out = pl.pallas_call(
        body,
        out_shape=jax.ShapeDtypeStruct((N, C2, M), jnp.float32),
        grid=(N,),
        in_specs=[
            pl.BlockSpec((None, C1, M), lambda n: (n, 0, 0)),
            _full_spec(wub.shape), _full_spec(w_ra.shape),
            _full_spec(wrb.shape), _full_spec(w_tie.shape),
            _full_spec(sb.shape), _full_spec(sc.shape),
        ],
        out_specs=pl.BlockSpec((None, C2, M), lambda n: (n, 0, 0)),
        compiler_params=pltpu.CompilerParams(
            dimension_semantics=("parallel",)),
    )(xf, wub, w_ra, wrb, w_tie, sb, sc)

    return out.reshape(N, C2, H, W)
